# Initial kernel scaffold; baseline (speedup 1.0000x reference)
import jax
import jax.numpy as jnp
from jax.experimental import pallas as pl


def kernel(x, table, W, b):
    raise NotImplementedError("write your pallas kernel here")


# speedup vs baseline: 1.3078x; 1.3078x over previous
import functools

import jax
import jax.numpy as jnp
from jax import lax
from jax.experimental import pallas as pl
from jax.experimental.pallas import tpu as pltpu
from jax.experimental.pallas import tpu_sc as plsc

D = 1000
B_TOTAL = 81920
NC = 2
NS = 16
NW = NC * NS
B_PER_W = B_TOTAL // NW
CHUNK = 40
N_CHUNKS = B_PER_W // CHUNK


def _mm_body(t_ref, w_ref, b_ref, y_ref):
    y_ref[...] = lax.dot_general(
        t_ref[...], w_ref[...],
        dimension_numbers=(((1,), (1,)), ((), ())),
        preferred_element_type=jnp.float32,
    ) + b_ref[...]


def _fused_table(table, W, b):
    return pl.pallas_call(
        _mm_body,
        out_shape=jax.ShapeDtypeStruct((D, D), jnp.float32),
    )(table, W, b.reshape(1, D))


_sc_mesh = plsc.VectorSubcoreMesh(
    core_axis_name="c", subcore_axis_name="s", num_cores=NC, num_subcores=NS
)


@functools.partial(
    pl.kernel,
    out_type=jax.ShapeDtypeStruct((B_TOTAL, D), jnp.float32),
    mesh=_sc_mesh,
    scratch_types=[
        pltpu.VMEM((B_PER_W,), jnp.int32),
        pltpu.VMEM((CHUNK, D), jnp.float32),
        pltpu.SemaphoreType.DMA,
    ],
    compiler_params=pltpu.CompilerParams(use_tc_tiling_on_sc=False),
)
def _sc_gather(y_hbm, idx_hbm, out_hbm, idx_v, rows_v, sem):
    wid = lax.axis_index("s") * NC + lax.axis_index("c")
    base = wid * B_PER_W
    pltpu.sync_copy(idx_hbm.at[pl.ds(base, B_PER_W)], idx_v)

    def body(c, _):
        start = c * CHUNK
        pltpu.async_copy(y_hbm.at[idx_v.at[pl.ds(start, CHUNK)]], rows_v, sem).wait()
        pltpu.sync_copy(rows_v, out_hbm.at[pl.ds(base + start, CHUNK)])
        return 0

    lax.fori_loop(0, N_CHUNKS, body, 0)


def kernel(x, table, W, b):
    y = _fused_table(table, W, b)
    idx = x.reshape(-1).astype(jnp.int32)
    out = _sc_gather(y, idx)
    return out.reshape(x.shape[0], x.shape[1], D)


# speedup vs baseline: 1.3605x; 1.0403x over previous
import functools

import jax
import jax.numpy as jnp
from jax import lax
from jax.experimental import pallas as pl
from jax.experimental.pallas import tpu as pltpu
from jax.experimental.pallas import tpu_sc as plsc

D = 1000
B_TOTAL = 81920
NC = 2
NS = 16
NW = NC * NS
B_PER_W = B_TOTAL // NW
CHUNK = 64
N_CHUNKS = B_PER_W // CHUNK


def _mm_body(t_ref, w_ref, b_ref, y_ref):
    y_ref[...] = lax.dot_general(
        t_ref[...], w_ref[...],
        dimension_numbers=(((1,), (1,)), ((), ())),
        preferred_element_type=jnp.float32,
    ) + b_ref[...]


def _fused_table(table, W, b):
    return pl.pallas_call(
        _mm_body,
        out_shape=jax.ShapeDtypeStruct((D, D), jnp.float32),
    )(table, W, b.reshape(1, D))


_sc_mesh = plsc.VectorSubcoreMesh(
    core_axis_name="c", subcore_axis_name="s", num_cores=NC, num_subcores=NS
)


@functools.partial(
    pl.kernel,
    out_type=jax.ShapeDtypeStruct((B_TOTAL, D), jnp.float32),
    mesh=_sc_mesh,
    scratch_types=[
        pltpu.VMEM((B_PER_W,), jnp.int32),
        pltpu.VMEM((CHUNK, D), jnp.float32),
        pltpu.VMEM((CHUNK, D), jnp.float32),
        pltpu.SemaphoreType.DMA,
        pltpu.SemaphoreType.DMA,
        pltpu.SemaphoreType.DMA,
        pltpu.SemaphoreType.DMA,
    ],
    compiler_params=pltpu.CompilerParams(use_tc_tiling_on_sc=False),
)
def _sc_gather(y_hbm, idx_hbm, out_hbm, idx_v, rows_a, rows_b,
               gsem_a, gsem_b, ssem_a, ssem_b):
    wid = lax.axis_index("s") * NC + lax.axis_index("c")
    base = wid * B_PER_W
    pltpu.sync_copy(idx_hbm.at[pl.ds(base, B_PER_W)], idx_v)

    rows = (rows_a, rows_b)
    gsem = (gsem_a, gsem_b)
    ssem = (ssem_a, ssem_b)

    def g_start(c, s):
        pltpu.make_async_copy(
            y_hbm.at[idx_v.at[pl.ds(c * CHUNK, CHUNK)]], rows[s], gsem[s]
        ).start()

    def g_wait(s):
        pltpu.make_async_copy(
            y_hbm.at[idx_v.at[pl.ds(0, CHUNK)]], rows[s], gsem[s]
        ).wait()

    def s_start(c, s):
        pltpu.make_async_copy(
            rows[s], out_hbm.at[pl.ds(base + c * CHUNK, CHUNK)], ssem[s]
        ).start()

    def s_wait(s):
        pltpu.make_async_copy(
            rows[s], out_hbm.at[pl.ds(base, CHUNK)], ssem[s]
        ).wait()

    g_start(0, 0)

    def pair(p, _):
        for s in range(2):
            c = 2 * p + s
            g_wait(s)
            o = 1 - s

            @pl.when(c >= 1)
            def _():
                s_wait(o)

            @pl.when(c + 1 < N_CHUNKS)
            def _():
                g_start(c + 1, o)

            s_start(c, s)
        return 0

    lax.fori_loop(0, N_CHUNKS // 2, pair, 0)
    s_wait((N_CHUNKS - 1) % 2)


def kernel(x, table, W, b):
    y = _fused_table(table, W, b)
    idx = x.reshape(-1).astype(jnp.int32)
    out = _sc_gather(y, idx)
    return out.reshape(x.shape[0], x.shape[1], D)
